# Initial kernel scaffold; baseline (speedup 1.0000x reference)
import jax
import jax.numpy as jnp
from jax.experimental import pallas as pl


def kernel(inputs, embeddings):
    raise NotImplementedError("write your pallas kernel here")


# speedup vs baseline: 6.0509x; 6.0509x over previous
import functools

import jax
import jax.numpy as jnp
from jax.experimental import pallas as pl

V = 8
N = 16384
D = 64
K = 1024
BN = 1024


def _vq_body(x_ref, w_ref, wt_ref, out_ref):
    x = x_ref[0]
    w = w_ref[0]
    wt = wt_ref[0]
    scores = jnp.dot(x, w, preferred_element_type=jnp.float32)
    xsq = jnp.sum(x * x, axis=1, keepdims=True)
    wsq = jnp.sum(w * w, axis=0, keepdims=True)
    dist = xsq - 2.0 * scores + wsq
    m = jnp.min(dist, axis=1, keepdims=True)
    iota = jax.lax.broadcasted_iota(jnp.int32, (BN, K), 1)
    idx = jnp.min(jnp.where(dist == m, iota, K), axis=1, keepdims=True)
    onehot = (iota == idx).astype(jnp.float32)
    q = jnp.dot(onehot, wt, preferred_element_type=jnp.float32)
    out_ref[0] = x + (q - x)


@jax.jit
def _vq_tc(inputs, embeddings, emb_t):
    grid = (V, N // BN)
    return pl.pallas_call(
        _vq_body,
        grid=grid,
        in_specs=[
            pl.BlockSpec((1, BN, D), lambda v, n: (v, n, 0)),
            pl.BlockSpec((1, D, K), lambda v, n: (v, 0, 0)),
            pl.BlockSpec((1, K, D), lambda v, n: (v, 0, 0)),
        ],
        out_specs=pl.BlockSpec((1, BN, D), lambda v, n: (v, n, 0)),
        out_shape=jax.ShapeDtypeStruct((V, N, D), jnp.float32),
    )(inputs, embeddings, emb_t)


def kernel(inputs, embeddings):
    emb_t = jnp.transpose(embeddings, (0, 2, 1))
    return _vq_tc(inputs, embeddings, emb_t)
